# baseline (speedup 1.0000x reference)
import functools

import jax
import jax.numpy as jnp
from jax.experimental import pallas as pl

E = 320000
T = 640000
H = 128
INT = 64

_RB = 2000
_TB = 4000


def _act(v):
    return v * jax.nn.sigmoid(v)


def _dot(a, b):
    return jax.lax.dot_general(a, b, (((1,), (0,)), ((), ())),
                               preferred_element_type=jnp.float32)


def _stage_a_body(x1, rbfg_w, gji_w, gji_b, gkj_w, gkj_b, gdown,
                  xjig_o, xkd_o):
    x = x1[...]
    xjig_o[...] = _act(_dot(x, gji_w[...]) + gji_b[...])
    xk = _act(_dot(x, gkj_w[...]) + gkj_b[...])
    xk = xk * rbfg_w[...]
    xkd_o[...] = _act(_dot(xk, gdown[...]))


def _stage_a(x1, rbfg, p):
    nb = E // _RB
    full = lambda r, c: pl.BlockSpec((r, c), lambda i: (0, 0))
    blk = lambda c: pl.BlockSpec((_RB, c), lambda i: (i, 0))
    return pl.pallas_call(
        _stage_a_body,
        grid=(nb,),
        in_specs=[blk(H), blk(H), full(H, H), full(1, H), full(H, H),
                  full(1, H), full(H, INT)],
        out_specs=[blk(H), blk(INT)],
        out_shape=[jax.ShapeDtypeStruct((E, H), jnp.float32),
                   jax.ShapeDtypeStruct((E, INT), jnp.float32)],
    )(x1, rbfg, p["g_ji_w"], p["g_ji_b"].reshape(1, H), p["g_kj_w"],
      p["g_kj_b"].reshape(1, H), p["g_down"])


def _stage_c_body(agg1, xjig, x1, rbf, gup, w1, b1, w2, b2, skw, skb, qdown,
                  qmpg_o, xqd_o):
    x_kj_g = _act(_dot(agg1[...], gup[...]))
    qmpg = xjig[...] + x_kj_g
    h = _act(_dot(qmpg, w1[...]) + b1[...])
    qmpg = qmpg + _act(_dot(h, w2[...]) + b2[...])
    qmpg_o[...] = _act(_dot(qmpg, skw[...]) + skb[...]) + x1[...]
    xq = x_kj_g * rbf[...]
    xqd_o[...] = _act(_dot(xq, qdown[...]))


def _stage_c(agg1, xjig, x1, rbf, p):
    nb = E // _RB
    full = lambda r, c: pl.BlockSpec((r, c), lambda i: (0, 0))
    blk = lambda c: pl.BlockSpec((_RB, c), lambda i: (i, 0))
    (w1, b1, w2, b2), = p["res_before"]
    return pl.pallas_call(
        _stage_c_body,
        grid=(nb,),
        in_specs=[blk(INT), blk(H), blk(H), blk(H), full(INT, H),
                  full(H, H), full(1, H), full(H, H), full(1, H),
                  full(H, H), full(1, H), full(H, INT)],
        out_specs=[blk(H), blk(INT)],
        out_shape=[jax.ShapeDtypeStruct((E, H), jnp.float32),
                   jax.ShapeDtypeStruct((E, INT), jnp.float32)],
    )(agg1, xjig, x1, rbf, p["g_up"], w1, b1.reshape(1, H), w2,
      b2.reshape(1, H), p["skip_w"], p["skip_b"].reshape(1, H), p["q_down"])


def _stage_d_body(sbf, t, sbf1, sbf2, t1, t2, st_o):
    sb = _dot(_dot(sbf[...], sbf1[...]), sbf2[...])
    tt = _dot(_dot(t[...], t1[...]), t2[...])
    st_o[...] = sb * tt


def _stage_d(sbf, t, p):
    nb = T // _TB
    full = lambda r, c: pl.BlockSpec((r, c), lambda i: (0, 0))
    return pl.pallas_call(
        _stage_d_body,
        grid=(nb,),
        in_specs=[pl.BlockSpec((_TB, 18), lambda i: (i, 0)),
                  pl.BlockSpec((_TB, 54), lambda i: (i, 0)),
                  full(18, 8), full(8, INT), full(54, INT), full(INT, INT)],
        out_specs=pl.BlockSpec((_TB, INT), lambda i: (i, 0)),
        out_shape=jax.ShapeDtypeStruct((T, INT), jnp.float32),
    )(sbf, t, p["q_sbf1"], p["q_sbf2"], p["q_t1"], p["q_t2"])


def _stage_f_body(agg2, qmpg, rl, qup, linw, linb, aw1, ab1, aw2, ab2,
                  aw3, ab3, aw4, ab4, e1_o, e2_o):
    qmpq = _act(_dot(agg2[...], qup[...]))
    e2 = _act(_dot(qmpg[...] + qmpq, linw[...]) + linb[...])
    h = _act(_dot(e2, aw1[...]) + ab1[...])
    e2 = e2 + _act(_dot(h, aw2[...]) + ab2[...])
    h = _act(_dot(e2, aw3[...]) + ab3[...])
    e2 = e2 + _act(_dot(h, aw4[...]) + ab4[...])
    e2_o[...] = e2
    e1_o[...] = rl[...] * e2


def _stage_f(agg2, qmpg, rl, p):
    nb = E // _RB
    full = lambda r, c: pl.BlockSpec((r, c), lambda i: (0, 0))
    blk = lambda c: pl.BlockSpec((_RB, c), lambda i: (i, 0))
    (aw1, ab1, aw2, ab2), (aw3, ab3, aw4, ab4) = p["res_after"]
    return pl.pallas_call(
        _stage_f_body,
        grid=(nb,),
        in_specs=[blk(INT), blk(H), blk(H), full(INT, H), full(H, H),
                  full(1, H), full(H, H), full(1, H), full(H, H), full(1, H),
                  full(H, H), full(1, H), full(H, H), full(1, H)],
        out_specs=[blk(H), blk(H)],
        out_shape=[jax.ShapeDtypeStruct((E, H), jnp.float32),
                   jax.ShapeDtypeStruct((E, H), jnp.float32)],
    )(agg2, qmpg, rl, p["q_up"], p["lin_w"], p["lin_b"].reshape(1, H),
      aw1, ab1.reshape(1, H), aw2, ab2.reshape(1, H),
      aw3, ab3.reshape(1, H), aw4, ab4.reshape(1, H))


def _rbf_body(rbf, w, o):
    o[...] = _dot(rbf[...], w[...])


def _rbf_proj(rbf, w):
    nb = E // _RB
    return pl.pallas_call(
        _rbf_body,
        grid=(nb,),
        in_specs=[pl.BlockSpec((_RB, 6), lambda i: (i, 0)),
                  pl.BlockSpec((6, H), lambda i: (0, 0))],
        out_specs=pl.BlockSpec((_RB, H), lambda i: (i, 0)),
        out_shape=jax.ShapeDtypeStruct((E, H), jnp.float32),
    )(rbf, w)


def kernel(x1, x2, rbf0, sbf, t, rbf0_g, params, idx_kj, idx_ji):
    p = params
    g_rbf12 = p["g_rbf1"] @ p["g_rbf2"]
    q_rbf12 = p["q_rbf1"] @ p["q_rbf2"]

    rbfg = _rbf_proj(rbf0_g, g_rbf12)
    rbf = _rbf_proj(rbf0, q_rbf12)
    rl = _rbf_proj(rbf0, p["lin_rbf"])

    xjig, xkd = _stage_a(x1, rbfg, p)

    g1 = xkd[idx_kj]
    agg1 = jnp.zeros((E, INT), jnp.float32).at[idx_ji].add(g1)

    qmpg, xqd = _stage_c(agg1, xjig, x1, rbf, p)

    st = _stage_d(sbf, t, p)

    g2 = xqd[idx_kj] * st
    agg2 = jnp.zeros((E, INT), jnp.float32).at[idx_ji].add(g2)

    e1, e2 = _stage_f(agg2, qmpg, rl, p)
    return (e1, e2)


# speedup vs baseline: 1.0359x; 1.0359x over previous
import functools

import jax
import jax.numpy as jnp
from jax.experimental import pallas as pl

E = 320000
T = 640000
H = 128
INT = 64

_RB = 2000
_TB = 4000


def _act(v):
    return v * jax.nn.sigmoid(v)


def _dot(a, b):
    return jax.lax.dot_general(a, b, (((1,), (0,)), ((), ())),
                               preferred_element_type=jnp.float32)


def _stage_a_body(x1, rbf0g, grbf12, gji_w, gji_b, gkj_w, gkj_b, gdown,
                  xjig_o, xkd_o):
    x = x1[...]
    rbfg = _dot(rbf0g[...], grbf12[...])
    xjig_o[...] = _act(_dot(x, gji_w[...]) + gji_b[...])
    xk = _act(_dot(x, gkj_w[...]) + gkj_b[...])
    xk = xk * rbfg
    xkd_o[...] = _act(_dot(xk, gdown[...]))


def _stage_a(x1, rbf0_g, g_rbf12, p):
    nb = E // _RB
    full = lambda r, c: pl.BlockSpec((r, c), lambda i: (0, 0))
    blk = lambda c: pl.BlockSpec((_RB, c), lambda i: (i, 0))
    return pl.pallas_call(
        _stage_a_body,
        grid=(nb,),
        in_specs=[blk(H), blk(6), full(6, H), full(H, H), full(1, H),
                  full(H, H), full(1, H), full(H, INT)],
        out_specs=[blk(H), blk(INT)],
        out_shape=[jax.ShapeDtypeStruct((E, H), jnp.float32),
                   jax.ShapeDtypeStruct((E, INT), jnp.float32)],
    )(x1, rbf0_g, g_rbf12, p["g_ji_w"], p["g_ji_b"].reshape(1, H),
      p["g_kj_w"], p["g_kj_b"].reshape(1, H), p["g_down"])


def _stage_c_body(agg1, xjig, x1, rbf0, qrbf12, gup, w1, b1, w2, b2, skw,
                  skb, qdown, qmpg_o, xqd_o):
    rbf = _dot(rbf0[...], qrbf12[...])
    x_kj_g = _act(_dot(agg1[...], gup[...]))
    qmpg = xjig[...] + x_kj_g
    h = _act(_dot(qmpg, w1[...]) + b1[...])
    qmpg = qmpg + _act(_dot(h, w2[...]) + b2[...])
    qmpg_o[...] = _act(_dot(qmpg, skw[...]) + skb[...]) + x1[...]
    xq = x_kj_g * rbf
    xqd_o[...] = _act(_dot(xq, qdown[...]))


def _stage_c(agg1, xjig, x1, rbf0, q_rbf12, p):
    nb = E // _RB
    full = lambda r, c: pl.BlockSpec((r, c), lambda i: (0, 0))
    blk = lambda c: pl.BlockSpec((_RB, c), lambda i: (i, 0))
    (w1, b1, w2, b2), = p["res_before"]
    return pl.pallas_call(
        _stage_c_body,
        grid=(nb,),
        in_specs=[blk(INT), blk(H), blk(H), blk(6), full(6, H),
                  full(INT, H), full(H, H), full(1, H), full(H, H),
                  full(1, H), full(H, H), full(1, H), full(H, INT)],
        out_specs=[blk(H), blk(INT)],
        out_shape=[jax.ShapeDtypeStruct((E, H), jnp.float32),
                   jax.ShapeDtypeStruct((E, INT), jnp.float32)],
    )(agg1, xjig, x1, rbf0, q_rbf12, p["g_up"], w1, b1.reshape(1, H), w2,
      b2.reshape(1, H), p["skip_w"], p["skip_b"].reshape(1, H), p["q_down"])


def _stage_d_body(sbf, t, sbf12, t12, st_o):
    sb = _dot(sbf[...], sbf12[...])
    tt = _dot(t[...], t12[...])
    st_o[...] = sb * tt


def _stage_d(sbf, t, sbf12, t12):
    nb = T // _TB
    full = lambda r, c: pl.BlockSpec((r, c), lambda i: (0, 0))
    return pl.pallas_call(
        _stage_d_body,
        grid=(nb,),
        in_specs=[pl.BlockSpec((_TB, 18), lambda i: (i, 0)),
                  pl.BlockSpec((_TB, 54), lambda i: (i, 0)),
                  full(18, INT), full(54, INT)],
        out_specs=pl.BlockSpec((_TB, INT), lambda i: (i, 0)),
        out_shape=jax.ShapeDtypeStruct((T, INT), jnp.float32),
    )(sbf, t, sbf12, t12)


def _stage_f_body(agg2, qmpg, rbf0, linrbf, qup, linw, linb, aw1, ab1, aw2,
                  ab2, aw3, ab3, aw4, ab4, e1_o, e2_o):
    rl = _dot(rbf0[...], linrbf[...])
    qmpq = _act(_dot(agg2[...], qup[...]))
    e2 = _act(_dot(qmpg[...] + qmpq, linw[...]) + linb[...])
    h = _act(_dot(e2, aw1[...]) + ab1[...])
    e2 = e2 + _act(_dot(h, aw2[...]) + ab2[...])
    h = _act(_dot(e2, aw3[...]) + ab3[...])
    e2 = e2 + _act(_dot(h, aw4[...]) + ab4[...])
    e2_o[...] = e2
    e1_o[...] = rl * e2


def _stage_f(agg2, qmpg, rbf0, p):
    nb = E // _RB
    full = lambda r, c: pl.BlockSpec((r, c), lambda i: (0, 0))
    blk = lambda c: pl.BlockSpec((_RB, c), lambda i: (i, 0))
    (aw1, ab1, aw2, ab2), (aw3, ab3, aw4, ab4) = p["res_after"]
    return pl.pallas_call(
        _stage_f_body,
        grid=(nb,),
        in_specs=[blk(INT), blk(H), blk(6), full(6, H), full(INT, H),
                  full(H, H), full(1, H), full(H, H), full(1, H),
                  full(H, H), full(1, H), full(H, H), full(1, H),
                  full(H, H), full(1, H)],
        out_specs=[blk(H), blk(H)],
        out_shape=[jax.ShapeDtypeStruct((E, H), jnp.float32),
                   jax.ShapeDtypeStruct((E, H), jnp.float32)],
    )(agg2, qmpg, rbf0, p["lin_rbf"], p["q_up"], p["lin_w"],
      p["lin_b"].reshape(1, H), aw1, ab1.reshape(1, H), aw2,
      ab2.reshape(1, H), aw3, ab3.reshape(1, H), aw4, ab4.reshape(1, H))


def kernel(x1, x2, rbf0, sbf, t, rbf0_g, params, idx_kj, idx_ji):
    p = params
    g_rbf12 = p["g_rbf1"] @ p["g_rbf2"]
    q_rbf12 = p["q_rbf1"] @ p["q_rbf2"]
    sbf12 = p["q_sbf1"] @ p["q_sbf2"]
    t12 = p["q_t1"] @ p["q_t2"]

    xjig, xkd = _stage_a(x1, rbf0_g, g_rbf12, p)

    g1 = xkd[idx_kj]
    agg1 = jnp.zeros((E, INT), jnp.float32).at[idx_ji].add(g1)

    qmpg, xqd = _stage_c(agg1, xjig, x1, rbf0, q_rbf12, p)

    st = _stage_d(sbf, t, sbf12, t12)

    g2 = xqd[idx_kj] * st
    agg2 = jnp.zeros((E, INT), jnp.float32).at[idx_ji].add(g2)

    e1, e2 = _stage_f(agg2, qmpg, rbf0, p)
    return (e1, e2)
